# baseline (speedup 1.0000x reference)
import functools

import jax
import jax.numpy as jnp
from jax import lax
from jax.experimental import pallas as pl
from jax.experimental.pallas import tpu as pltpu
from jax.experimental.pallas import tpu_sc as plsc

_NC = 2
_NS = 16
_NW = _NC * _NS
_K = 16
_IPS = 64
_GPC = _IPS // _K
_NBUF = 4


def _rne_bf16_bits(v):
    u = lax.bitcast_convert_type(v, jnp.uint32)
    return (u + jnp.uint32(0x7FFF)
            + ((u >> jnp.uint32(16)) & jnp.uint32(1))) >> jnp.uint32(16)


def _pack_halves(y):
    d = y.shape[-1]
    lo = _rne_bf16_bits(y[:, : d // 2])
    hi = _rne_bf16_bits(y[:, d // 2:])
    return (hi << jnp.uint32(16)) | lo


def _gather_sum_body(table_hbm, nbr_hbm, out_hbm, idx_v, buf0, buf1, buf2,
                     buf3, acc0, acc1, sem0, sem1, sem2, sem3, osem0, osem1,
                     *, dp, cw, rw):
    wid = lax.axis_index("s") * _NC + lax.axis_index("c")
    bufs = (buf0, buf1, buf2, buf3)
    accs = (acc0, acc1)
    sems = (sem0, sem1, sem2, sem3)
    osems = (osem0, osem1)
    pltpu.sync_copy(nbr_hbm.at[pl.ds(wid * cw, cw), :], idx_v)

    def reduce_chunk(b, c):
        buf, acc = bufs[b], accs[b % 2]

        def per_node(g, _):
            base = g * _K
            for col in range(dp // 16):
                cs = pl.ds(col * 16, 16)
                words = [buf[base + r, cs] for r in range(_K)]

                lo = [lax.bitcast_convert_type(w << jnp.uint32(16),
                                               jnp.float32) for w in words]
                hi = [lax.bitcast_convert_type(w, jnp.float32) for w in words]
                while len(lo) > 1:
                    lo = [lo[2 * j] + lo[2 * j + 1] for j in range(len(lo) // 2)]
                    hi = [hi[2 * j] + hi[2 * j + 1] for j in range(len(hi) // 2)]
                lo_u = lax.bitcast_convert_type(lo[0], jnp.uint32)
                hi_u = lax.bitcast_convert_type(hi[0], jnp.uint32)
                acc[g, cs] = ((hi_u & jnp.uint32(0xFFFF0000))
                              | (lo_u >> jnp.uint32(16)))
            return 0

        lax.fori_loop(0, _GPC, per_node, 0)
        pltpu.async_copy(acc,
                         out_hbm.at[pl.ds(wid * rw + c * _GPC, _GPC), :],
                         osems[b % 2])

    def wait_out(b):
        pltpu.make_async_copy(accs[b],
                              out_hbm.at[pl.ds(wid * rw, _GPC), :],
                              osems[b]).wait()

    def gather(c, b):
        pltpu.async_copy(table_hbm.at[idx_v.at[c]], bufs[b], sems[b])

    def wait_gather(c, b):
        pltpu.make_async_copy(table_hbm.at[idx_v.at[c]], bufs[b],
                              sems[b]).wait()

    for p in range(_NBUF - 1):
        gather(p, p)

    def ring(i, _):
        for b in range(_NBUF):
            c = i * _NBUF + b
            wait_gather(c, b)

            @pl.when(c + _NBUF - 1 < cw)
            def _():
                gather(c + _NBUF - 1, (b + _NBUF - 1) % _NBUF)

            if b >= 2:
                wait_out(b % 2)
            else:
                @pl.when(i > 0)
                def _():
                    wait_out(b % 2)

            reduce_chunk(b, c)
        return 0

    lax.fori_loop(0, cw // _NBUF, ring, 0)
    wait_out(0)
    wait_out(1)


def _gather_sum(table_pk, nbr_flat, n_pad):
    dp = table_pk.shape[1]
    rw = n_pad // _NW
    cw = rw // _GPC
    nbr_blocks = nbr_flat.reshape(-1, _IPS)
    mesh = plsc.VectorSubcoreMesh(core_axis_name="c", subcore_axis_name="s")
    body = functools.partial(_gather_sum_body, dp=dp, cw=cw, rw=rw)
    return pl.kernel(
        body,
        mesh=mesh,
        out_type=jax.ShapeDtypeStruct((n_pad, dp), jnp.uint32),
        scratch_types=(
            [pltpu.VMEM((cw, _IPS), jnp.int32)]
            + [pltpu.VMEM((_IPS, dp), jnp.uint32) for _ in range(_NBUF)]
            + [pltpu.VMEM((_GPC, dp), jnp.uint32) for _ in range(2)]
            + [pltpu.SemaphoreType.DMA for _ in range(_NBUF + 2)]
        ),
        name=f"sage_gather_sum_d{2 * dp}",
    )(table_pk, nbr_blocks)


def _pack_feat_body(x_ref, bf_ref, pk_ref):
    x = x_ref[...]
    bf_ref[...] = x.astype(jnp.bfloat16)
    pk_ref[...] = _pack_halves(x)


def _pack_feat(feat, bm):
    n, d = feat.shape
    return pl.pallas_call(
        _pack_feat_body,
        grid=(n // bm,),
        in_specs=[pl.BlockSpec((bm, d), lambda i: (i, 0))],
        out_specs=[
            pl.BlockSpec((bm, d), lambda i: (i, 0)),
            pl.BlockSpec((bm, d // 2), lambda i: (i, 0)),
        ],
        out_shape=[
            jax.ShapeDtypeStruct((n, d), jnp.bfloat16),
            jax.ShapeDtypeStruct((n, d // 2), jnp.uint32),
        ],
        name="sage_pack_feat",
    )(feat)


def _combine_body(feat_ref, agg_ref, w_ref, *out_refs, d_in, inv_k, pack):
    x = feat_ref[...]
    pk = agg_ref[...]
    alo = lax.bitcast_convert_type(pk << jnp.uint32(16), jnp.float32)
    ahi = lax.bitcast_convert_type(pk & jnp.uint32(0xFFFF0000), jnp.float32)
    a = jnp.concatenate([alo, ahi], axis=1).astype(jnp.bfloat16)
    ws = w_ref[:, :d_in]
    wn = w_ref[:, d_in:]
    y = lax.dot_general(x, ws, (((1,), (1,)), ((), ())),
                        preferred_element_type=jnp.float32)
    y = y + inv_k * lax.dot_general(a, wn, (((1,), (1,)), ((), ())),
                                    preferred_element_type=jnp.float32)
    y = jnp.maximum(y, 0.0)
    out_refs[0][...] = y.astype(out_refs[0].dtype)
    if pack:
        out_refs[1][...] = _pack_halves(y)


def _combine(feat, agg_sum, w, bm, out_dtype, pack):
    n, d_in = feat.shape
    d_out = w.shape[0]
    body = functools.partial(_combine_body, d_in=d_in, inv_k=1.0 / _K,
                             pack=pack)
    out_specs = [pl.BlockSpec((bm, d_out), lambda i: (i, 0))]
    out_shape = [jax.ShapeDtypeStruct((n, d_out), out_dtype)]
    if pack:
        out_specs.append(pl.BlockSpec((bm, d_out // 2), lambda i: (i, 0)))
        out_shape.append(jax.ShapeDtypeStruct((n, d_out // 2), jnp.uint32))
    res = pl.pallas_call(
        body,
        grid=(n // bm,),
        in_specs=[
            pl.BlockSpec((bm, d_in), lambda i: (i, 0)),
            pl.BlockSpec((bm, d_in // 2), lambda i: (i, 0)),
            pl.BlockSpec((d_out, 2 * d_in), lambda i: (0, 0)),
        ],
        out_specs=out_specs if pack else out_specs[0],
        out_shape=out_shape if pack else out_shape[0],
        name=f"sage_combine_{d_in}",
    )(feat, agg_sum, w)
    return res


def kernel(nodes, feat_data, neighbors, W0, W1):
    del nodes
    n, d_in = feat_data.shape

    quantum = _NW * _GPC * _NBUF
    n_pad = ((n + quantum - 1) // quantum) * quantum
    pad = n_pad - n
    nbr = neighbors.astype(jnp.int32).reshape(-1)
    if pad:
        dummy = (jnp.arange(pad * _K, dtype=jnp.int32) * 97) % n
        nbr = jnp.concatenate([nbr, dummy])

    bm = 1000 if n % 1000 == 0 else max(
        b for b in (512, 400, 256, 200, 128, 100, 80, 50, 40, 25, 20, 16, 10, 8, 5, 4, 2, 1)
        if n % b == 0)

    w0_bf = W0.astype(jnp.bfloat16)
    w1_bf = W1.astype(jnp.bfloat16)

    feat_bf, feat_pk = _pack_feat(feat_data, bm)
    agg0 = _gather_sum(feat_pk, nbr, n_pad)
    h1, h1_pk = _combine(feat_bf, agg0, w0_bf, bm, jnp.bfloat16, pack=True)
    agg1 = _gather_sum(h1_pk, nbr, n_pad)
    return _combine(h1, agg1, w1_bf, bm, jnp.float32, pack=False)


# speedup vs baseline: 1.0527x; 1.0527x over previous
import functools

import jax
import jax.numpy as jnp
from jax import lax
from jax.experimental import pallas as pl
from jax.experimental.pallas import tpu as pltpu
from jax.experimental.pallas import tpu_sc as plsc

_NC = 2
_NS = 16
_NW = _NC * _NS
_K = 16
_IPS = 128
_GPC = _IPS // _K
_NBUF = 3


def _rne_bf16_bits(v):
    u = lax.bitcast_convert_type(v, jnp.uint32)
    return (u + jnp.uint32(0x7FFF)
            + ((u >> jnp.uint32(16)) & jnp.uint32(1))) >> jnp.uint32(16)


def _pack_halves(y):
    d = y.shape[-1]
    lo = _rne_bf16_bits(y[:, : d // 2])
    hi = _rne_bf16_bits(y[:, d // 2:])
    return (hi << jnp.uint32(16)) | lo


def _gather_sum_body(table_hbm, nbr_hbm, out_hbm, idx_v, buf0, buf1, buf2,
                     acc0, acc1, sem0, sem1, sem2, osem0, osem1,
                     *, dp, cw, rw):
    wid = lax.axis_index("s") * _NC + lax.axis_index("c")
    bufs = (buf0, buf1, buf2)
    accs = (acc0, acc1)
    sems = (sem0, sem1, sem2)
    osems = (osem0, osem1)
    pltpu.sync_copy(nbr_hbm.at[wid], idx_v)

    def reduce_chunk(b, c):
        buf, acc = bufs[b], accs[b % 2]

        def per_node(g, _):
            base = g * _K
            for col in range(dp // 16):
                cs = pl.ds(col * 16, 16)
                words = [buf[base + r, cs] for r in range(_K)]

                lo = [lax.bitcast_convert_type(w << jnp.uint32(16),
                                               jnp.float32) for w in words]
                hi = [lax.bitcast_convert_type(w, jnp.float32) for w in words]
                while len(lo) > 1:
                    lo = [lo[2 * j] + lo[2 * j + 1] for j in range(len(lo) // 2)]
                    hi = [hi[2 * j] + hi[2 * j + 1] for j in range(len(hi) // 2)]
                lo_u = lax.bitcast_convert_type(lo[0], jnp.uint32)
                hi_u = lax.bitcast_convert_type(hi[0], jnp.uint32)
                acc[g, cs] = ((hi_u & jnp.uint32(0xFFFF0000))
                              | (lo_u >> jnp.uint32(16)))
            return 0

        lax.fori_loop(0, _GPC, per_node, 0)
        pltpu.async_copy(acc,
                         out_hbm.at[pl.ds(wid * rw + c * _GPC, _GPC), :],
                         osems[b % 2])

    def wait_out(b):
        pltpu.make_async_copy(accs[b],
                              out_hbm.at[pl.ds(wid * rw, _GPC), :],
                              osems[b]).wait()

    def gather(c, b):
        pltpu.async_copy(table_hbm.at[idx_v.at[c]], bufs[b], sems[b])

    def wait_gather(c, b):
        pltpu.make_async_copy(table_hbm.at[idx_v.at[c]], bufs[b],
                              sems[b]).wait()

    for p in range(_NBUF - 1):
        gather(p, p)

    def ring(i, _):
        for b in range(_NBUF):
            c = i * _NBUF + b
            wait_gather(c, b)

            @pl.when(c + _NBUF - 1 < cw)
            def _():
                gather(c + _NBUF - 1, (b + _NBUF - 1) % _NBUF)

            if b >= 2:
                wait_out(b % 2)
            else:
                @pl.when(i > 0)
                def _():
                    wait_out(b % 2)

            reduce_chunk(b, c)
        return 0

    lax.fori_loop(0, cw // _NBUF, ring, 0)
    wait_out(0)
    wait_out(1)


def _gather_sum(table_pk, nbr_flat, n_pad):
    dp = table_pk.shape[1]
    rw = n_pad // _NW
    cw = rw // _GPC
    nbr_blocks = nbr_flat.reshape(_NW, cw, _IPS)
    mesh = plsc.VectorSubcoreMesh(core_axis_name="c", subcore_axis_name="s")
    body = functools.partial(_gather_sum_body, dp=dp, cw=cw, rw=rw)
    return pl.kernel(
        body,
        mesh=mesh,
        out_type=jax.ShapeDtypeStruct((n_pad, dp), jnp.uint32),
        scratch_types=(
            [pltpu.VMEM((cw, _IPS), jnp.int32)]
            + [pltpu.VMEM((_IPS, dp), jnp.uint32) for _ in range(_NBUF)]
            + [pltpu.VMEM((_GPC, dp), jnp.uint32) for _ in range(2)]
            + [pltpu.SemaphoreType.DMA for _ in range(_NBUF + 2)]
        ),
        name=f"sage_gather_sum_d{2 * dp}",
    )(table_pk, nbr_blocks)


def _pack_feat_body(x_ref, bf_ref, pk_ref):
    x = x_ref[...]
    bf_ref[...] = x.astype(jnp.bfloat16)
    pk_ref[...] = _pack_halves(x)


def _pack_feat(feat, bm):
    n, d = feat.shape
    return pl.pallas_call(
        _pack_feat_body,
        grid=(n // bm,),
        in_specs=[pl.BlockSpec((bm, d), lambda i: (i, 0))],
        out_specs=[
            pl.BlockSpec((bm, d), lambda i: (i, 0)),
            pl.BlockSpec((bm, d // 2), lambda i: (i, 0)),
        ],
        out_shape=[
            jax.ShapeDtypeStruct((n, d), jnp.bfloat16),
            jax.ShapeDtypeStruct((n, d // 2), jnp.uint32),
        ],
        name="sage_pack_feat",
    )(feat)


def _combine_body(feat_ref, agg_ref, w_ref, *out_refs, d_in, inv_k, pack):
    x = feat_ref[...]
    pk = agg_ref[...]
    alo = lax.bitcast_convert_type(pk << jnp.uint32(16), jnp.float32)
    ahi = lax.bitcast_convert_type(pk & jnp.uint32(0xFFFF0000), jnp.float32)
    a = jnp.concatenate([alo, ahi], axis=1).astype(jnp.bfloat16)
    ws = w_ref[:, :d_in]
    wn = w_ref[:, d_in:]
    y = lax.dot_general(x, ws, (((1,), (1,)), ((), ())),
                        preferred_element_type=jnp.float32)
    y = y + inv_k * lax.dot_general(a, wn, (((1,), (1,)), ((), ())),
                                    preferred_element_type=jnp.float32)
    y = jnp.maximum(y, 0.0)
    out_refs[0][...] = y.astype(out_refs[0].dtype)
    if pack:
        out_refs[1][...] = _pack_halves(y)


def _combine(feat, agg_sum, w, bm, out_dtype, pack):
    n, d_in = feat.shape
    d_out = w.shape[0]
    body = functools.partial(_combine_body, d_in=d_in, inv_k=1.0 / _K,
                             pack=pack)
    out_specs = [pl.BlockSpec((bm, d_out), lambda i: (i, 0))]
    out_shape = [jax.ShapeDtypeStruct((n, d_out), out_dtype)]
    if pack:
        out_specs.append(pl.BlockSpec((bm, d_out // 2), lambda i: (i, 0)))
        out_shape.append(jax.ShapeDtypeStruct((n, d_out // 2), jnp.uint32))
    res = pl.pallas_call(
        body,
        grid=(n // bm,),
        in_specs=[
            pl.BlockSpec((bm, d_in), lambda i: (i, 0)),
            pl.BlockSpec((bm, d_in // 2), lambda i: (i, 0)),
            pl.BlockSpec((d_out, 2 * d_in), lambda i: (0, 0)),
        ],
        out_specs=out_specs if pack else out_specs[0],
        out_shape=out_shape if pack else out_shape[0],
        name=f"sage_combine_{d_in}",
    )(feat, agg_sum, w)
    return res


def kernel(nodes, feat_data, neighbors, W0, W1):
    del nodes
    n, d_in = feat_data.shape

    quantum = _NW * _GPC * _NBUF
    n_pad = ((n + quantum - 1) // quantum) * quantum
    pad = n_pad - n
    nbr = neighbors.astype(jnp.int32).reshape(-1)
    if pad:
        dummy = (jnp.arange(pad * _K, dtype=jnp.int32) * 97) % n
        nbr = jnp.concatenate([nbr, dummy])

    bm = 1000 if n % 1000 == 0 else max(
        b for b in (512, 400, 256, 200, 128, 100, 80, 50, 40, 25, 20, 16, 10, 8, 5, 4, 2, 1)
        if n % b == 0)

    w0_bf = W0.astype(jnp.bfloat16)
    w1_bf = W1.astype(jnp.bfloat16)

    feat_bf, feat_pk = _pack_feat(feat_data, bm)
    agg0 = _gather_sum(feat_pk, nbr, n_pad)
    h1, h1_pk = _combine(feat_bf, agg0, w0_bf, bm, jnp.bfloat16, pack=True)
    agg1 = _gather_sum(h1_pk, nbr, n_pad)
    return _combine(h1, agg1, w1_bf, bm, jnp.float32, pack=False)


# speedup vs baseline: 1.1242x; 1.0678x over previous
import functools

import jax
import jax.numpy as jnp
from jax import lax
from jax.experimental import pallas as pl
from jax.experimental.pallas import tpu as pltpu
from jax.experimental.pallas import tpu_sc as plsc

_NC = 2
_NS = 16
_NW = _NC * _NS
_K = 16
_IPS = 128
_GPC = _IPS // _K
_NBUF = 2


def _rne_bf16_bits(v):
    u = lax.bitcast_convert_type(v, jnp.uint32)
    return (u + jnp.uint32(0x7FFF)
            + ((u >> jnp.uint32(16)) & jnp.uint32(1))) >> jnp.uint32(16)


def _pack_halves(y):
    d = y.shape[-1]
    lo = _rne_bf16_bits(y[:, : d // 2])
    hi = _rne_bf16_bits(y[:, d // 2:])
    return (hi << jnp.uint32(16)) | lo


def _gather_sum_body(table_hbm, nbr_hbm, out_hbm, *scratch, dp, cw, rw):
    wid = lax.axis_index("s") * _NC + lax.axis_index("c")
    idx_v = scratch[0]
    bufs = scratch[1:1 + _NBUF]
    accs = scratch[1 + _NBUF:3 + _NBUF]
    sems = scratch[3 + _NBUF:3 + 2 * _NBUF]
    osems = scratch[3 + 2 * _NBUF:5 + 2 * _NBUF]
    pltpu.sync_copy(nbr_hbm.at[wid], idx_v)

    def reduce_chunk(b, c):
        buf, acc = bufs[b], accs[b % 2]

        def per_node(g, _):
            base = g * _K
            for col in range(dp // 16):
                cs = pl.ds(col * 16, 16)
                words = [buf[base + r, cs] for r in range(_K)]

                lo = [lax.bitcast_convert_type(w << jnp.uint32(16),
                                               jnp.float32) for w in words]
                hi = [lax.bitcast_convert_type(w, jnp.float32) for w in words]
                while len(lo) > 1:
                    lo = [lo[2 * j] + lo[2 * j + 1] for j in range(len(lo) // 2)]
                    hi = [hi[2 * j] + hi[2 * j + 1] for j in range(len(hi) // 2)]
                lo_u = lax.bitcast_convert_type(lo[0], jnp.uint32)
                hi_u = lax.bitcast_convert_type(hi[0], jnp.uint32)
                acc[g, cs] = ((hi_u & jnp.uint32(0xFFFF0000))
                              | (lo_u >> jnp.uint32(16)))
            return 0

        lax.fori_loop(0, _GPC, per_node, 0)
        pltpu.async_copy(acc,
                         out_hbm.at[pl.ds(wid * rw + c * _GPC, _GPC), :],
                         osems[b % 2])

    def wait_out(b):
        pltpu.make_async_copy(accs[b],
                              out_hbm.at[pl.ds(wid * rw, _GPC), :],
                              osems[b]).wait()

    def gather(c, b):
        pltpu.async_copy(table_hbm.at[idx_v.at[c]], bufs[b], sems[b])

    def wait_gather(c, b):
        pltpu.make_async_copy(table_hbm.at[idx_v.at[c]], bufs[b],
                              sems[b]).wait()

    for p in range(_NBUF - 1):
        gather(p, p)

    def ring(i, _):
        for b in range(_NBUF):
            c = i * _NBUF + b
            wait_gather(c, b)

            @pl.when(c + _NBUF - 1 < cw)
            def _():
                gather(c + _NBUF - 1, (b + _NBUF - 1) % _NBUF)

            if b >= 2:
                wait_out(b % 2)
            else:
                @pl.when(i > 0)
                def _():
                    wait_out(b % 2)

            reduce_chunk(b, c)
        return 0

    lax.fori_loop(0, cw // _NBUF, ring, 0)
    wait_out(0)
    wait_out(1)


def _gather_sum(table_pk, nbr_flat, n_pad):
    dp = table_pk.shape[1]
    rw = n_pad // _NW
    cw = rw // _GPC
    nbr_blocks = nbr_flat.reshape(_NW, cw, _IPS)
    mesh = plsc.VectorSubcoreMesh(core_axis_name="c", subcore_axis_name="s")
    body = functools.partial(_gather_sum_body, dp=dp, cw=cw, rw=rw)
    return pl.kernel(
        body,
        mesh=mesh,
        out_type=jax.ShapeDtypeStruct((n_pad, dp), jnp.uint32),
        scratch_types=(
            [pltpu.VMEM((cw, _IPS), jnp.int32)]
            + [pltpu.VMEM((_IPS, dp), jnp.uint32) for _ in range(_NBUF)]
            + [pltpu.VMEM((_GPC, dp), jnp.uint32) for _ in range(2)]
            + [pltpu.SemaphoreType.DMA for _ in range(_NBUF + 2)]
        ),
        name=f"sage_gather_sum_d{2 * dp}",
    )(table_pk, nbr_blocks)


def _pack_feat_body(x_ref, bf_ref, pk_ref):
    x = x_ref[...]
    bf_ref[...] = x.astype(jnp.bfloat16)
    pk_ref[...] = _pack_halves(x)


def _pack_feat(feat, bm):
    n, d = feat.shape
    return pl.pallas_call(
        _pack_feat_body,
        grid=(n // bm,),
        in_specs=[pl.BlockSpec((bm, d), lambda i: (i, 0))],
        out_specs=[
            pl.BlockSpec((bm, d), lambda i: (i, 0)),
            pl.BlockSpec((bm, d // 2), lambda i: (i, 0)),
        ],
        out_shape=[
            jax.ShapeDtypeStruct((n, d), jnp.bfloat16),
            jax.ShapeDtypeStruct((n, d // 2), jnp.uint32),
        ],
        name="sage_pack_feat",
    )(feat)


def _combine_body(feat_ref, agg_ref, w_ref, *out_refs, d_in, inv_k, pack):
    x = feat_ref[...]
    pk = agg_ref[...]
    alo = lax.bitcast_convert_type(pk << jnp.uint32(16), jnp.float32)
    ahi = lax.bitcast_convert_type(pk & jnp.uint32(0xFFFF0000), jnp.float32)
    a = jnp.concatenate([alo, ahi], axis=1).astype(jnp.bfloat16)
    ws = w_ref[:, :d_in]
    wn = w_ref[:, d_in:]
    y = lax.dot_general(x, ws, (((1,), (1,)), ((), ())),
                        preferred_element_type=jnp.float32)
    y = y + inv_k * lax.dot_general(a, wn, (((1,), (1,)), ((), ())),
                                    preferred_element_type=jnp.float32)
    y = jnp.maximum(y, 0.0)
    out_refs[0][...] = y.astype(out_refs[0].dtype)
    if pack:
        out_refs[1][...] = _pack_halves(y)


def _combine(feat, agg_sum, w, bm, out_dtype, pack):
    n, d_in = feat.shape
    d_out = w.shape[0]
    body = functools.partial(_combine_body, d_in=d_in, inv_k=1.0 / _K,
                             pack=pack)
    out_specs = [pl.BlockSpec((bm, d_out), lambda i: (i, 0))]
    out_shape = [jax.ShapeDtypeStruct((n, d_out), out_dtype)]
    if pack:
        out_specs.append(pl.BlockSpec((bm, d_out // 2), lambda i: (i, 0)))
        out_shape.append(jax.ShapeDtypeStruct((n, d_out // 2), jnp.uint32))
    res = pl.pallas_call(
        body,
        grid=(n // bm,),
        in_specs=[
            pl.BlockSpec((bm, d_in), lambda i: (i, 0)),
            pl.BlockSpec((bm, d_in // 2), lambda i: (i, 0)),
            pl.BlockSpec((d_out, 2 * d_in), lambda i: (0, 0)),
        ],
        out_specs=out_specs if pack else out_specs[0],
        out_shape=out_shape if pack else out_shape[0],
        name=f"sage_combine_{d_in}",
    )(feat, agg_sum, w)
    return res


def kernel(nodes, feat_data, neighbors, W0, W1):
    del nodes
    n, d_in = feat_data.shape

    quantum = _NW * _GPC * _NBUF
    n_pad = ((n + quantum - 1) // quantum) * quantum
    pad = n_pad - n
    nbr = neighbors.astype(jnp.int32).reshape(-1)
    if pad:
        dummy = (jnp.arange(pad * _K, dtype=jnp.int32) * 97) % n
        nbr = jnp.concatenate([nbr, dummy])

    bm = 1000 if n % 1000 == 0 else max(
        b for b in (512, 400, 256, 200, 128, 100, 80, 50, 40, 25, 20, 16, 10, 8, 5, 4, 2, 1)
        if n % b == 0)

    w0_bf = W0.astype(jnp.bfloat16)
    w1_bf = W1.astype(jnp.bfloat16)

    feat_bf, feat_pk = _pack_feat(feat_data, bm)
    agg0 = _gather_sum(feat_pk, nbr, n_pad)
    h1, h1_pk = _combine(feat_bf, agg0, w0_bf, bm, jnp.bfloat16, pack=True)
    agg1 = _gather_sum(h1_pk, nbr, n_pad)
    return _combine(h1, agg1, w1_bf, bm, jnp.float32, pack=False)
